# baseline (speedup 1.0000x reference)
import functools

import jax
import jax.numpy as jnp
from jax import lax
from jax.experimental import pallas as pl
from jax.experimental.pallas import tpu as pltpu
from jax.experimental.pallas import tpu_sc as plsc

D_EMBED = 64
SCALE = float(D_EMBED ** 0.5)

IDX_W = 128
ROWS = 4096 * 200 // IDX_W
NW = 32
ROWS_PER_W = ROWS // NW
K = 2
N_CHUNKS = ROWS_PER_W // K

_mesh = plsc.VectorSubcoreMesh(core_axis_name="c", subcore_axis_name="s")


@functools.partial(
    pl.kernel,
    mesh=_mesh,
    out_type=jax.ShapeDtypeStruct((ROWS, IDX_W, D_EMBED), jnp.float32),
    scratch_types=[
        pltpu.VMEM((ROWS_PER_W, IDX_W), jnp.int32),
        pltpu.VMEM((2 * K, IDX_W, D_EMBED), jnp.float32),
        pltpu.SemaphoreType.DMA,
    ],
    compiler_params=pltpu.CompilerParams(use_tc_tiling_on_sc=False),
)
def _gather_scale(idx_hbm, table_hbm, out_hbm, idx_v, rows_v, sem):
    wid = lax.axis_index("s") * 2 + lax.axis_index("c")
    row0 = wid * ROWS_PER_W
    pltpu.sync_copy(idx_hbm.at[pl.ds(row0, ROWS_PER_W)], idx_v)

    def fire(c, b):
        for j in range(K):
            pltpu.async_copy(
                table_hbm.at[idx_v.at[c * K + j]], rows_v.at[b * K + j], sem)

    def process(c, b):
        for j in range(K):
            pltpu.make_async_copy(
                table_hbm.at[idx_v.at[c * K + j]], rows_v.at[b * K + j], sem
            ).wait()

        def scale_body(rr, carry):
            for kk in range(K):
                for j in range(D_EMBED // 16):
                    sl = (b * K + kk, rr, pl.ds(j * 16, 16))
                    rows_v[sl] = rows_v[sl] * SCALE
            return carry

        lax.fori_loop(0, IDX_W, scale_body, 0)
        pltpu.sync_copy(rows_v.at[pl.ds(b * K, K)],
                        out_hbm.at[pl.ds(row0 + c * K, K)])

    fire(0, 0)

    def pair_body(i, carry):
        c0 = 2 * i
        fire(c0 + 1, 1)
        process(c0, 0)
        fire(c0 + 2, 0)
        process(c0 + 1, 1)
        return carry

    lax.fori_loop(0, (N_CHUNKS - 2) // 2, pair_body, 0)
    fire(N_CHUNKS - 1, 1)
    process(N_CHUNKS - 2, 0)
    process(N_CHUNKS - 1, 1)


def kernel(inp, emb_weight):
    idx = inp.reshape(ROWS, IDX_W)
    if idx.dtype != jnp.int32:
        idx = idx.astype(jnp.int32)
    out = _gather_scale(idx, emb_weight)
    return out.reshape(inp.shape[0], inp.shape[1], D_EMBED)


# speedup vs baseline: 1.0099x; 1.0099x over previous
import functools

import jax
import jax.numpy as jnp
from jax import lax
from jax.experimental import pallas as pl
from jax.experimental.pallas import tpu as pltpu
from jax.experimental.pallas import tpu_sc as plsc

D_EMBED = 64
SCALE = float(D_EMBED ** 0.5)

IDX_W = 128
ROWS = 4096 * 200 // IDX_W
NW = 32
ROWS_PER_W = ROWS // NW
K = 5
N_CHUNKS = ROWS_PER_W // K

_mesh = plsc.VectorSubcoreMesh(core_axis_name="c", subcore_axis_name="s")


@functools.partial(
    pl.kernel,
    mesh=_mesh,
    out_type=jax.ShapeDtypeStruct((ROWS, IDX_W, D_EMBED), jnp.float32),
    scratch_types=[
        pltpu.VMEM((ROWS_PER_W, IDX_W), jnp.int32),
        pltpu.VMEM((2 * K, IDX_W, D_EMBED), jnp.float32),
        pltpu.SemaphoreType.DMA,
    ],
    compiler_params=pltpu.CompilerParams(use_tc_tiling_on_sc=False),
)
def _gather_scale(idx_hbm, table_hbm, out_hbm, idx_v, rows_v, sem):
    wid = lax.axis_index("s") * 2 + lax.axis_index("c")
    row0 = wid * ROWS_PER_W
    pltpu.sync_copy(idx_hbm.at[pl.ds(row0, ROWS_PER_W)], idx_v)

    def fire(c, b):
        for j in range(K):
            pltpu.async_copy(
                table_hbm.at[idx_v.at[c * K + j]], rows_v.at[b * K + j], sem)

    def process(c, b):
        for j in range(K):
            pltpu.make_async_copy(
                table_hbm.at[idx_v.at[c * K + j]], rows_v.at[b * K + j], sem
            ).wait()

        def scale_body(rr, carry):
            for kk in range(K):
                for j in range(D_EMBED // 16):
                    sl = (b * K + kk, rr, pl.ds(j * 16, 16))
                    rows_v[sl] = rows_v[sl] * SCALE
            return carry

        lax.fori_loop(0, IDX_W, scale_body, 0)
        pltpu.sync_copy(rows_v.at[pl.ds(b * K, K)],
                        out_hbm.at[pl.ds(row0 + c * K, K)])

    fire(0, 0)

    def pair_body(i, carry):
        c0 = 2 * i
        fire(c0 + 1, 1)
        process(c0, 0)
        fire(c0 + 2, 0)
        process(c0 + 1, 1)
        return carry

    lax.fori_loop(0, (N_CHUNKS - 2) // 2, pair_body, 0)
    fire(N_CHUNKS - 1, 1)
    process(N_CHUNKS - 2, 0)
    process(N_CHUNKS - 1, 1)


def kernel(inp, emb_weight):
    idx = inp.reshape(ROWS, IDX_W)
    if idx.dtype != jnp.int32:
        idx = idx.astype(jnp.int32)
    out = _gather_scale(idx, emb_weight)
    return out.reshape(inp.shape[0], inp.shape[1], D_EMBED)
